# baseline (speedup 1.0000x reference)
import functools

import jax
import jax.numpy as jnp
from jax import lax
from jax.experimental import pallas as pl
from jax.experimental.pallas import tpu as pltpu
from jax.experimental.pallas import tpu_sc as plsc

NC = 2
NS = 16
NW = NC * NS
L = 16

B = 16384
K = 32
B_PER_W = B // NW
N_SUB = 4
SUB = B_PER_W // N_SUB


def _body(u_hbm, i_hbm, p_hbm, q_hbm, out_hbm,
          ui_v, ii_v, pu_v, qi_v, out_v, sem_p, sem_q):
    wid = lax.axis_index("s") * NC + lax.axis_index("c")
    base = wid * B_PER_W

    pltpu.sync_copy(u_hbm.at[wid], ui_v)
    pltpu.sync_copy(i_hbm.at[wid], ii_v)

    copies = []
    for j in range(N_SUB):
        rows = pl.ds(j * SUB, SUB)
        copies.append(pltpu.async_copy(p_hbm.at[ui_v.at[j]], pu_v.at[rows], sem_p))
        copies.append(pltpu.async_copy(q_hbm.at[ii_v.at[j]], qi_v.at[rows], sem_q))
    for c in copies:
        c.wait()

    def chunk(c, carry):
        rows = c * L + lax.iota(jnp.int32, L)
        acc = jnp.zeros((L,), jnp.float32)
        for k in range(K):
            col = jnp.full((L,), k, jnp.int32)
            acc = acc + (plsc.load_gather(pu_v, [rows, col]) *
                         plsc.load_gather(qi_v, [rows, col]))
        out_v[pl.ds(c * L, L)] = acc
        return carry

    lax.fori_loop(0, B_PER_W // L, chunk, 0, unroll=False)

    pltpu.sync_copy(out_v, out_hbm.at[pl.ds(base, B_PER_W)])


@jax.jit
def _funk_svd_sc(u2, i2, P, Q):
    mesh = plsc.VectorSubcoreMesh(core_axis_name="c", subcore_axis_name="s")
    return pl.kernel(
        _body,
        out_type=jax.ShapeDtypeStruct((B,), jnp.float32),
        mesh=mesh,
        scratch_types=[
            pltpu.VMEM((N_SUB, SUB), jnp.int32),
            pltpu.VMEM((N_SUB, SUB), jnp.int32),
            pltpu.VMEM((B_PER_W, K), jnp.float32),
            pltpu.VMEM((B_PER_W, K), jnp.float32),
            pltpu.VMEM((B_PER_W,), jnp.float32),
            pltpu.SemaphoreType.DMA,
            pltpu.SemaphoreType.DMA,
        ],
        compiler_params=pltpu.CompilerParams(
            needs_layout_passes=False, use_tc_tiling_on_sc=False),
    )(u2, i2, P, Q)


def kernel(u, i, P, Q):
    u2 = u.astype(jnp.int32).reshape(NW, N_SUB, SUB)
    i2 = i.astype(jnp.int32).reshape(NW, N_SUB, SUB)
    return _funk_svd_sc(u2, i2, P, Q)
